# Initial kernel scaffold; baseline (speedup 1.0000x reference)
import jax
import jax.numpy as jnp
from jax.experimental import pallas as pl


def kernel(x_emb, w1, b1, w2, b2, wfc, bfc):
    raise NotImplementedError("write your pallas kernel here")


# speedup vs baseline: 1.3754x; 1.3754x over previous
import functools

import jax
import jax.numpy as jnp
from jax.experimental import pallas as pl
from jax.experimental.pallas import tpu as pltpu

K1, S1 = 10, 5
K2, S2 = 10, 7
LANE = 128


def _round_up(x, m):
    return (x + m - 1) // m * m


def _make_body(nd):
    def body(x_ref, w1_ref, b1_ref, w2_ref, b2_ref, wfc_ref, bfc_ref, out_ref):
        xb = x_ref[...].astype(jnp.bfloat16)
        h = jnp.dot(xb, w1_ref[...], preferred_element_type=jnp.float32)
        h = jnp.maximum(h + b1_ref[...], 0.0).astype(jnp.bfloat16)
        o2 = jnp.dot(h, w2_ref[...], preferred_element_type=jnp.float32)
        o2 = jnp.maximum(o2 + b2_ref[...], 0.0).astype(jnp.bfloat16)
        out_ref[...] = (jnp.dot(o2, wfc_ref[...], preferred_element_type=jnp.float32)
                        + bfc_ref[...])
    return body


@functools.partial(jax.jit, static_argnames=("batch_block",))
def _impl(x_emb, w1, b1, w2, b2, wfc, bfc, *, batch_block=512):
    B, L, E = x_emb.shape
    H = w1.shape[0]
    C2 = w2.shape[0]
    n_cls = wfc.shape[0]
    L1 = (L - K1) // S1 + 1
    L2 = (L1 - K2) // S2 + 1
    T1 = (L2 - 1) * S2 + K2
    XW = L * E
    HTOT = _round_up(T1 * H, LANE)
    C2P = _round_up(C2, LANE)
    OUTW = LANE
    cdtype = jnp.bfloat16

    w1f = jnp.transpose(w1, (2, 1, 0)).reshape(K1 * E, H)
    rows = jnp.arange(XW)[:, None] - S1 * E * jnp.arange(T1)[None, :]
    valid = (rows >= 0) & (rows < K1 * E)
    g = jnp.take(w1f, jnp.clip(rows, 0, K1 * E - 1), axis=0)
    W1 = (g * valid[..., None]).reshape(XW, T1 * H)
    W1 = jnp.pad(W1, ((0, 0), (0, HTOT - T1 * H))).astype(cdtype)
    b1b = jnp.pad(jnp.tile(b1.astype(jnp.float32), T1), (0, HTOT - T1 * H))
    b1b = b1b.reshape(1, HTOT)

    w2t = jnp.transpose(w2, (2, 1, 0))
    kk = jnp.arange(T1)[:, None] - S2 * jnp.arange(L2)[None, :]
    vk = (kk >= 0) & (kk < K2)
    g2 = jnp.take(w2t, jnp.clip(kk, 0, K2 - 1), axis=0)
    g2 = g2 * vk[..., None, None]
    g2 = jnp.transpose(g2, (0, 2, 1, 3))
    g2 = jnp.pad(g2, ((0, 0), (0, 0), (0, 0), (0, C2P - C2)))
    W2 = g2.reshape(T1 * H, L2 * C2P)
    W2 = jnp.pad(W2, ((0, HTOT - T1 * H), (0, 0))).astype(cdtype)
    b2b = jnp.tile(jnp.pad(b2.astype(jnp.float32), (0, C2P - C2)), L2)
    b2b = b2b.reshape(1, L2 * C2P)

    wfc_r = jnp.transpose(wfc.reshape(n_cls, C2, L2), (2, 1, 0))
    Wfc = jnp.pad(wfc_r, ((0, 0), (0, C2P - C2), (0, OUTW - n_cls)))
    Wfc = Wfc.reshape(L2 * C2P, OUTW).astype(cdtype)
    bfcb = jnp.pad(bfc.astype(jnp.float32), (0, OUTW - n_cls)).reshape(1, OUTW)

    BB = min(batch_block, _round_up(B, 8))
    nb = pl.cdiv(B, BB)
    Bp = nb * BB
    x2 = x_emb.reshape(B, XW)
    if Bp != B:
        x2 = jnp.pad(x2, ((0, Bp - B), (0, 0)))

    out = pl.pallas_call(
        _make_body(nb),
        out_shape=jax.ShapeDtypeStruct((Bp, OUTW), jnp.float32),
        grid=(nb,),
        in_specs=[
            pl.BlockSpec((BB, XW), lambda i: (i, 0)),
            pl.BlockSpec((XW, HTOT), lambda i: (0, 0)),
            pl.BlockSpec((1, HTOT), lambda i: (0, 0)),
            pl.BlockSpec((HTOT, L2 * C2P), lambda i: (0, 0)),
            pl.BlockSpec((1, L2 * C2P), lambda i: (0, 0)),
            pl.BlockSpec((L2 * C2P, OUTW), lambda i: (0, 0)),
            pl.BlockSpec((1, OUTW), lambda i: (0, 0)),
        ],
        out_specs=pl.BlockSpec((BB, OUTW), lambda i: (i, 0)),
        compiler_params=pltpu.CompilerParams(
            dimension_semantics=("parallel",),
            vmem_limit_bytes=64 * 1024 * 1024,
        ),
    )(x2, W1, b1b, W2, b2b, Wfc, bfcb)

    return out[:B, :n_cls]


def kernel(x_emb, w1, b1, w2, b2, wfc, bfc):
    return _impl(x_emb, w1, b1, w2, b2, wfc, bfc)


# speedup vs baseline: 1.8308x; 1.3310x over previous
import functools

import jax
import jax.numpy as jnp
from jax.experimental import pallas as pl
from jax.experimental.pallas import tpu as pltpu

K1, S1 = 10, 5
K2, S2 = 10, 7
LANE = 128


def _round_up(x, m):
    return (x + m - 1) // m * m


def _make_body(nd):
    def body(x_ref, w1_ref, b1_ref, w2_ref, b2_ref, wfc_ref, bfc_ref, out_ref):
        xb = x_ref[...].astype(jnp.bfloat16)
        h = jnp.dot(xb, w1_ref[...], preferred_element_type=jnp.float32)
        h = jnp.maximum(h + b1_ref[...], 0.0).astype(jnp.bfloat16)
        o2 = jnp.dot(h, w2_ref[...], preferred_element_type=jnp.float32)
        o2 = jnp.maximum(o2 + b2_ref[...], 0.0).astype(jnp.bfloat16)
        out_ref[...] = (jnp.dot(o2, wfc_ref[...], preferred_element_type=jnp.float32)
                        + bfc_ref[...])
    return body


@functools.partial(jax.jit, static_argnames=("batch_block",))
def _impl(x_emb, w1, b1, w2, b2, wfc, bfc, *, batch_block=512):
    B, L, E = x_emb.shape
    H = w1.shape[0]
    C2 = w2.shape[0]
    n_cls = wfc.shape[0]
    L1 = (L - K1) // S1 + 1
    L2 = (L1 - K2) // S2 + 1
    T1 = (L2 - 1) * S2 + K2
    XW = L * E
    HTOT = _round_up(T1 * H, LANE)
    C2P = _round_up(C2, LANE)
    OUTW = 8
    cdtype = jnp.bfloat16

    w1f = jnp.transpose(w1, (2, 1, 0)).reshape(K1 * E, H).astype(cdtype)
    W1 = jnp.concatenate(
        [jnp.pad(w1f, ((t * S1 * E, XW - K1 * E - t * S1 * E), (0, 0)))
         for t in range(T1)], axis=1)
    W1 = jnp.pad(W1, ((0, 0), (0, HTOT - T1 * H)))
    b1b = jnp.pad(jnp.tile(b1.astype(jnp.float32), T1), (0, HTOT - T1 * H))
    b1b = b1b.reshape(1, HTOT)

    w2f = jnp.transpose(w2, (2, 1, 0)).reshape(K2 * H, C2).astype(cdtype)
    W2 = jnp.concatenate(
        [jnp.pad(w2f, ((t2 * S2 * H, (T1 - t2 * S2 - K2) * H), (0, C2P - C2)))
         for t2 in range(L2)], axis=1)
    W2 = jnp.pad(W2, ((0, HTOT - T1 * H), (0, 0)))
    b2b = jnp.tile(jnp.pad(b2.astype(jnp.float32), (0, C2P - C2)), L2)
    b2b = b2b.reshape(1, L2 * C2P)

    wfc_r = jnp.transpose(wfc.reshape(n_cls, C2, L2), (2, 1, 0))
    Wfc = jnp.pad(wfc_r, ((0, 0), (0, C2P - C2), (0, OUTW - n_cls)))
    Wfc = Wfc.reshape(L2 * C2P, OUTW).astype(cdtype)
    bfcb = jnp.pad(bfc.astype(jnp.float32), (0, OUTW - n_cls)).reshape(1, OUTW)

    BB = min(batch_block, _round_up(B, 8))
    nb = pl.cdiv(B, BB)
    Bp = nb * BB
    x2 = x_emb.reshape(B, XW)
    if Bp != B:
        x2 = jnp.pad(x2, ((0, Bp - B), (0, 0)))

    out = pl.pallas_call(
        _make_body(nb),
        out_shape=jax.ShapeDtypeStruct((Bp, OUTW), jnp.float32),
        grid=(nb,),
        in_specs=[
            pl.BlockSpec((BB, XW), lambda i: (i, 0)),
            pl.BlockSpec((XW, HTOT), lambda i: (0, 0)),
            pl.BlockSpec((1, HTOT), lambda i: (0, 0)),
            pl.BlockSpec((HTOT, L2 * C2P), lambda i: (0, 0)),
            pl.BlockSpec((1, L2 * C2P), lambda i: (0, 0)),
            pl.BlockSpec((L2 * C2P, OUTW), lambda i: (0, 0)),
            pl.BlockSpec((1, OUTW), lambda i: (0, 0)),
        ],
        out_specs=pl.BlockSpec((BB, OUTW), lambda i: (i, 0)),
        compiler_params=pltpu.CompilerParams(
            dimension_semantics=("parallel",),
            vmem_limit_bytes=64 * 1024 * 1024,
        ),
    )(x2, W1, b1b, W2, b2b, Wfc, bfcb)

    return out[:B, :n_cls]


def kernel(x_emb, w1, b1, w2, b2, wfc, bfc):
    return _impl(x_emb, w1, b1, w2, b2, wfc, bfc)


# speedup vs baseline: 4.2499x; 2.3214x over previous
import functools

import jax
import jax.numpy as jnp
from jax.experimental import pallas as pl
from jax.experimental.pallas import tpu as pltpu

K1, S1 = 10, 5
K2, S2 = 10, 7
LANE = 128


def _round_up(x, m):
    return (x + m - 1) // m * m


def _make_body(HTOT, C2L, OUTW):
    def body(x_ref, w1_ref, w2_ref, wfc_ref, bias_ref, out_ref):
        b1c = bias_ref[:, 0:1]
        b2c = bias_ref[0:C2L, 1:2]
        bfcc = bias_ref[0:OUTW, 2:3]
        xb = x_ref[...].astype(jnp.bfloat16)
        h = jnp.dot(w1_ref[...], xb, preferred_element_type=jnp.float32)
        h = jnp.maximum(h + b1c, 0.0).astype(jnp.bfloat16)
        o2 = jnp.dot(w2_ref[...], h, preferred_element_type=jnp.float32)
        o2 = jnp.maximum(o2 + b2c, 0.0).astype(jnp.bfloat16)
        out_ref[...] = (jnp.dot(wfc_ref[...], o2,
                                preferred_element_type=jnp.float32) + bfcc)
    return body


@functools.partial(jax.jit, static_argnames=("batch_block",))
def _impl(x_emb, w1, b1, w2, b2, wfc, bfc, *, batch_block=1024):
    B, L, E = x_emb.shape
    H = w1.shape[0]
    C2 = w2.shape[0]
    n_cls = wfc.shape[0]
    L1 = (L - K1) // S1 + 1
    L2 = (L1 - K2) // S2 + 1
    T1 = (L2 - 1) * S2 + K2
    XW = L * E
    HTOT = _round_up(T1 * H, LANE)
    C2P = _round_up(C2, LANE)
    C2L = L2 * C2P
    OUTW = 8
    cdtype = jnp.bfloat16

    w1f = jnp.transpose(w1, (0, 2, 1)).reshape(H, K1 * E).astype(cdtype)
    W1T = jnp.concatenate(
        [jnp.pad(w1f, ((0, 0), (t * S1 * E, XW - K1 * E - t * S1 * E)))
         for t in range(T1)], axis=0)
    W1T = jnp.pad(W1T, ((0, HTOT - T1 * H), (0, 0)))

    w2f = jnp.transpose(w2, (0, 2, 1)).reshape(C2, K2 * H).astype(cdtype)
    W2T = jnp.concatenate(
        [jnp.pad(w2f, ((0, C2P - C2), (t2 * S2 * H, (T1 - t2 * S2 - K2) * H)))
         for t2 in range(L2)], axis=0)
    W2T = jnp.pad(W2T, ((0, 0), (0, HTOT - T1 * H)))

    wfc_r = jnp.transpose(wfc.reshape(n_cls, C2, L2), (0, 2, 1))
    WfcT = jnp.pad(wfc_r, ((0, 0), (0, 0), (0, C2P - C2))).reshape(n_cls, C2L)
    WfcT = jnp.pad(WfcT, ((0, OUTW - n_cls), (0, 0))).astype(cdtype)

    c0 = jnp.pad(jnp.tile(b1.astype(jnp.float32), T1), (0, HTOT - T1 * H))
    c1 = jnp.pad(jnp.tile(jnp.pad(b2.astype(jnp.float32), (0, C2P - C2)), L2),
                 (0, HTOT - C2L))
    c2 = jnp.pad(bfc.astype(jnp.float32), (0, HTOT - n_cls))
    biasmat = jnp.pad(jnp.stack([c0, c1, c2], axis=1), ((0, 0), (0, 5)))

    xt = jnp.transpose(x_emb, (1, 2, 0)).reshape(XW, B)
    BN = min(batch_block, _round_up(B, LANE))
    nb = pl.cdiv(B, BN)
    Bp = nb * BN
    if Bp != B:
        xt = jnp.pad(xt, ((0, 0), (0, Bp - B)))

    out = pl.pallas_call(
        _make_body(HTOT, C2L, OUTW),
        out_shape=jax.ShapeDtypeStruct((OUTW, Bp), jnp.float32),
        grid=(nb,),
        in_specs=[
            pl.BlockSpec((XW, BN), lambda i: (0, i)),
            pl.BlockSpec((HTOT, XW), lambda i: (0, 0)),
            pl.BlockSpec((C2L, HTOT), lambda i: (0, 0)),
            pl.BlockSpec((OUTW, C2L), lambda i: (0, 0)),
            pl.BlockSpec((HTOT, 8), lambda i: (0, 0)),
        ],
        out_specs=pl.BlockSpec((OUTW, BN), lambda i: (0, i)),
        compiler_params=pltpu.CompilerParams(
            dimension_semantics=("parallel",),
            vmem_limit_bytes=64 * 1024 * 1024,
        ),
    )(xt, W1T, W2T, WfcT, biasmat)

    return jnp.transpose(out[:n_cls, :B])


def kernel(x_emb, w1, b1, w2, b2, wfc, bfc):
    return _impl(x_emb, w1, b1, w2, b2, wfc, bfc)


# speedup vs baseline: 4.4831x; 1.0549x over previous
import functools

import jax
import jax.numpy as jnp
from jax.experimental import pallas as pl
from jax.experimental.pallas import tpu as pltpu

K1, S1 = 10, 5
K2, S2 = 10, 7
LANE = 128


def _round_up(x, m):
    return (x + m - 1) // m * m


def _make_body(HTOT, C2L, OUTW):
    def body(x_ref, w1_ref, w2_ref, wfc_ref, bias_ref, out_ref):
        b1c = bias_ref[:, 0:1]
        b2c = bias_ref[0:C2L, 1:2]
        bfcc = bias_ref[0:OUTW, 2:3]
        xb = x_ref[...].astype(jnp.bfloat16)
        h = jnp.dot(w1_ref[...], xb, preferred_element_type=jnp.float32)
        h = jnp.maximum(h + b1c, 0.0).astype(jnp.bfloat16)
        o2 = jnp.dot(w2_ref[...], h, preferred_element_type=jnp.float32)
        o2 = jnp.maximum(o2 + b2c, 0.0).astype(jnp.bfloat16)
        out_ref[...] = (jnp.dot(wfc_ref[...], o2,
                                preferred_element_type=jnp.float32) + bfcc)
    return body


@functools.partial(jax.jit, static_argnames=("batch_block",))
def _impl(x_emb, w1, b1, w2, b2, wfc, bfc, *, batch_block=2048):
    B, L, E = x_emb.shape
    H = w1.shape[0]
    C2 = w2.shape[0]
    n_cls = wfc.shape[0]
    L1 = (L - K1) // S1 + 1
    L2 = (L1 - K2) // S2 + 1
    T1 = (L2 - 1) * S2 + K2
    XW = L * E
    HTOT = _round_up(T1 * H, LANE)
    C2P = _round_up(C2, LANE)
    C2L = L2 * C2P
    OUTW = 8
    cdtype = jnp.bfloat16

    w1f = jnp.transpose(w1, (0, 2, 1)).reshape(H, K1 * E).astype(cdtype)
    W1T = jnp.concatenate(
        [jnp.pad(w1f, ((0, 0), (t * S1 * E, XW - K1 * E - t * S1 * E)))
         for t in range(T1)], axis=0)
    W1T = jnp.pad(W1T, ((0, HTOT - T1 * H), (0, 0)))

    w2f = jnp.transpose(w2, (0, 2, 1)).reshape(C2, K2 * H).astype(cdtype)
    W2T = jnp.concatenate(
        [jnp.pad(w2f, ((0, C2P - C2), (t2 * S2 * H, (T1 - t2 * S2 - K2) * H)))
         for t2 in range(L2)], axis=0)
    W2T = jnp.pad(W2T, ((0, 0), (0, HTOT - T1 * H)))

    wfc_r = jnp.transpose(wfc.reshape(n_cls, C2, L2), (0, 2, 1))
    WfcT = jnp.pad(wfc_r, ((0, 0), (0, 0), (0, C2P - C2))).reshape(n_cls, C2L)
    WfcT = jnp.pad(WfcT, ((0, OUTW - n_cls), (0, 0))).astype(cdtype)

    c0 = jnp.pad(jnp.tile(b1.astype(jnp.float32), T1), (0, HTOT - T1 * H))
    c1 = jnp.pad(jnp.tile(jnp.pad(b2.astype(jnp.float32), (0, C2P - C2)), L2),
                 (0, HTOT - C2L))
    c2 = jnp.pad(bfc.astype(jnp.float32), (0, HTOT - n_cls))
    biasmat = jnp.pad(jnp.stack([c0, c1, c2], axis=1), ((0, 0), (0, 5)))

    xt = jnp.transpose(x_emb, (1, 2, 0)).reshape(XW, B)
    BN = min(batch_block, _round_up(B, LANE))
    nb = pl.cdiv(B, BN)
    Bp = nb * BN
    if Bp != B:
        xt = jnp.pad(xt, ((0, 0), (0, Bp - B)))

    out = pl.pallas_call(
        _make_body(HTOT, C2L, OUTW),
        out_shape=jax.ShapeDtypeStruct((OUTW, Bp), jnp.float32),
        grid=(nb,),
        in_specs=[
            pl.BlockSpec((XW, BN), lambda i: (0, i)),
            pl.BlockSpec((HTOT, XW), lambda i: (0, 0)),
            pl.BlockSpec((C2L, HTOT), lambda i: (0, 0)),
            pl.BlockSpec((OUTW, C2L), lambda i: (0, 0)),
            pl.BlockSpec((HTOT, 8), lambda i: (0, 0)),
        ],
        out_specs=pl.BlockSpec((OUTW, BN), lambda i: (0, i)),
        compiler_params=pltpu.CompilerParams(
            dimension_semantics=("parallel",),
            vmem_limit_bytes=64 * 1024 * 1024,
        ),
    )(xt, W1T, W2T, WfcT, biasmat)

    return jnp.transpose(out[:n_cls, :B])


def kernel(x_emb, w1, b1, w2, b2, wfc, bfc):
    return _impl(x_emb, w1, b1, w2, b2, wfc, bfc)
